# Initial kernel scaffold; baseline (speedup 1.0000x reference)
import jax
import jax.numpy as jnp
from jax.experimental import pallas as pl


def kernel(x, edge_index, edge_attr, embed_w, edge_embed_w, ggnn_weight, gru_w_ih, gru_w_hh, gru_b_ih, gru_b_hh, gate_w, gate_b, lstm_w_ih, lstm_w_hh, lstm_b_ih, lstm_b_hh, dense_w, dense_b):
    raise NotImplementedError("write your pallas kernel here")


# speedup vs baseline: 4.6698x; 4.6698x over previous
import functools

import jax
import jax.numpy as jnp
from jax import lax
from jax.experimental import pallas as pl
from jax.experimental.pallas import tpu as pltpu
from jax.experimental.pallas import tpu_sc as plsc

NC = 2
NS = 16
CH = 80


def _sc_embed(embed_w, xi_pad, P, D):
  rpt = P // (NC * NS)
  nch = rpt // CH
  assert rpt % CH == 0
  mesh = plsc.VectorSubcoreMesh(core_axis_name="c", subcore_axis_name="s")

  @functools.partial(
      pl.kernel,
      out_type=jax.ShapeDtypeStruct((P, D), jnp.float32),
      mesh=mesh,
      scratch_types=[
          pltpu.VMEM((CH,), jnp.int32),
          pltpu.VMEM((CH, D), jnp.float32),
          pltpu.SemaphoreType.DMA,
      ],
  )
  def k(emb, xi, out, ibuf, rows, sem):
    w = lax.axis_index("s") * NC + lax.axis_index("c")
    base = w * rpt
    for j in range(nch):
      off = pl.multiple_of(base + j * CH, 8)
      pltpu.sync_copy(xi.at[pl.ds(off, CH)], ibuf)
      pltpu.async_copy(emb.at[ibuf], rows, sem).wait()
      pltpu.sync_copy(rows, out.at[pl.ds(off, CH)])

  return k(embed_w, xi_pad)


def _sc_scatter(m, src, dst, zrows_h, P, D):
  E = src.shape[0]
  ept = E // (NC * NS)
  iters = ept // CH
  assert ept % CH == 0
  rpt = P // NS
  mesh = plsc.VectorSubcoreMesh(core_axis_name="c", subcore_axis_name="s")

  @functools.partial(
      pl.kernel,
      out_type=jax.ShapeDtypeStruct((NC * P, D), jnp.float32),
      mesh=mesh,
      scratch_types=[
          pltpu.VMEM((CH,), jnp.int32),
          pltpu.VMEM((CH,), jnp.int32),
          pltpu.VMEM((CH, D), jnp.float32),
          pltpu.VMEM((CH, D), jnp.float32),
          pltpu.SemaphoreType.DMA,
          pltpu.VMEM_SHARED((P, D), jnp.float32),
      ],
  )
  def k(m_h, src_h, dst_h, z_h, out_h, sbuf, dbuf, rows, zrows, sem, agg):
    c = lax.axis_index("c")
    s = lax.axis_index("s")
    w = s * NC + c
    pltpu.sync_copy(z_h, zrows)
    for j in range(rpt // CH):
      zoff = pl.multiple_of(s * rpt + j * CH, 8)
      pltpu.sync_copy(zrows, agg.at[pl.ds(zoff, CH)])
    plsc.subcore_barrier()

    base = w * ept

    def body(i, carry):
      off = pl.multiple_of(base + i * CH, 8)
      pltpu.sync_copy(src_h.at[pl.ds(off, CH)], sbuf)
      pltpu.sync_copy(dst_h.at[pl.ds(off, CH)], dbuf)
      pltpu.async_copy(m_h.at[sbuf], rows, sem).wait()
      pltpu.sync_copy(rows, agg.at[dbuf], add=True)
      return carry

    lax.fori_loop(0, iters, body, 0)
    plsc.subcore_barrier()
    ooff = pl.multiple_of(c * P + s * rpt, 8)
    soff = pl.multiple_of(s * rpt, 8)
    pltpu.sync_copy(agg.at[pl.ds(soff, rpt)], out_h.at[pl.ds(ooff, rpt)])

  return k(m, src, dst, zrows_h)


def _tc_m0(h, w0, P, D, BN):
  def body(h_ref, w_ref, o_ref):
    o_ref[...] = jnp.dot(h_ref[...], w_ref[...],
                         preferred_element_type=jnp.float32)

  return pl.pallas_call(
      body,
      grid=(P // BN,),
      in_specs=[
          pl.BlockSpec((BN, D), lambda j: (j, 0)),
          pl.BlockSpec((D, D), lambda j: (0, 0)),
      ],
      out_specs=pl.BlockSpec((BN, D), lambda j: (j, 0)),
      out_shape=jax.ShapeDtypeStruct((P, D), jnp.float32),
  )(h, w0)


def _tc_step(part, h, w_ih, w_hh, b_ih, b_hh, w_next, P, D, BN):
  def body(p_ref, h_ref, wih_ref, whh_ref, bih_ref, bhh_ref, wn_ref,
           hn_ref, mn_ref):
    agg = p_ref[0] + p_ref[1]
    h = h_ref[...]
    gi = lax.dot_general(agg, wih_ref[...], (((1,), (1,)), ((), ())),
                         preferred_element_type=jnp.float32) + bih_ref[...]
    gh = lax.dot_general(h, whh_ref[...], (((1,), (1,)), ((), ())),
                         preferred_element_type=jnp.float32) + bhh_ref[...]
    r = jax.nn.sigmoid(gi[:, :D] + gh[:, :D])
    z = jax.nn.sigmoid(gi[:, D:2 * D] + gh[:, D:2 * D])
    n = jnp.tanh(gi[:, 2 * D:] + r * gh[:, 2 * D:])
    hn = (1.0 - z) * n + z * h
    hn_ref[...] = hn
    mn_ref[...] = jnp.dot(hn, wn_ref[...], preferred_element_type=jnp.float32)

  return pl.pallas_call(
      body,
      grid=(P // BN,),
      in_specs=[
          pl.BlockSpec((NC, BN, D), lambda j: (0, j, 0)),
          pl.BlockSpec((BN, D), lambda j: (j, 0)),
          pl.BlockSpec((3 * D, D), lambda j: (0, 0)),
          pl.BlockSpec((3 * D, D), lambda j: (0, 0)),
          pl.BlockSpec((1, 3 * D), lambda j: (0, 0)),
          pl.BlockSpec((1, 3 * D), lambda j: (0, 0)),
          pl.BlockSpec((D, D), lambda j: (0, 0)),
      ],
      out_specs=[
          pl.BlockSpec((BN, D), lambda j: (j, 0)),
          pl.BlockSpec((BN, D), lambda j: (j, 0)),
      ],
      out_shape=[
          jax.ShapeDtypeStruct((P, D), jnp.float32),
          jax.ShapeDtypeStruct((P, D), jnp.float32),
      ],
  )(part, h, w_ih, w_hh, b_ih, b_hh, w_next)


def _tc_pool_lstm(h, gate_w, gate_b, lstm_w_ih, lstm_b, P, D, BN, N, LSTM):
  G = P // BN

  def body(h_ref, gw_ref, gb_ref, lw_ref, lb_ref, o_ref, sv, s1):
    j = pl.program_id(0)
    h = h_ref[...]
    g = jnp.sum(h * gw_ref[...], axis=1, keepdims=True) + gb_ref[0, 0]
    g = jax.nn.sigmoid(g)
    row = j * BN + lax.broadcasted_iota(jnp.int32, (BN, 1), 0)
    w = jnp.where(row < N, jnp.exp(g), 0.0)
    pv = lax.dot_general(w, h, (((0,), (0,)), ((), ())),
                         preferred_element_type=jnp.float32)
    ps = jnp.sum(w)

    @pl.when(j == 0)
    def _():
      sv[...] = jnp.zeros_like(sv)
      s1[...] = jnp.zeros_like(s1)

    sv[...] += pv
    s1[...] += ps

    @pl.when(j == G - 1)
    def _():
      hg = sv[...] / s1[0, 0]
      gates = lax.dot_general(hg, lw_ref[...], (((1,), (1,)), ((), ())),
                              preferred_element_type=jnp.float32) + lb_ref[...]
      i_t = jax.nn.sigmoid(gates[:, :LSTM])
      g_t = jnp.tanh(gates[:, 2 * LSTM:3 * LSTM])
      o_t = jax.nn.sigmoid(gates[:, 3 * LSTM:])
      c_t = i_t * g_t
      o_ref[...] = o_t * jnp.tanh(c_t)

  return pl.pallas_call(
      body,
      grid=(G,),
      in_specs=[
          pl.BlockSpec((BN, D), lambda j: (j, 0)),
          pl.BlockSpec((1, D), lambda j: (0, 0)),
          pl.BlockSpec((1, 1), lambda j: (0, 0)),
          pl.BlockSpec((4 * LSTM, D), lambda j: (0, 0)),
          pl.BlockSpec((1, 4 * LSTM), lambda j: (0, 0)),
      ],
      out_specs=pl.BlockSpec((1, LSTM), lambda j: (0, 0)),
      out_shape=jax.ShapeDtypeStruct((1, LSTM), jnp.float32),
      scratch_shapes=[
          pltpu.VMEM((1, D), jnp.float32),
          pltpu.VMEM((1, 1), jnp.float32),
      ],
  )(h, gate_w, gate_b, lstm_w_ih, lstm_b)


def _tc_dense(ht, dense_w, dense_b, LSTM, V, BV):
  def body(x_ref, w_ref, b_ref, o_ref):
    o_ref[...] = lax.dot_general(
        x_ref[...], w_ref[...], (((1,), (1,)), ((), ())),
        preferred_element_type=jnp.float32) + b_ref[...]

  return pl.pallas_call(
      body,
      grid=(pl.cdiv(V, BV),),
      in_specs=[
          pl.BlockSpec((1, LSTM), lambda j: (0, 0)),
          pl.BlockSpec((BV, LSTM), lambda j: (j, 0)),
          pl.BlockSpec((1, BV), lambda j: (0, j)),
      ],
      out_specs=pl.BlockSpec((1, BV), lambda j: (0, j)),
      out_shape=jax.ShapeDtypeStruct((1, V), jnp.float32),
  )(ht, dense_w, dense_b)


def kernel(x, edge_index, edge_attr, embed_w, edge_embed_w, ggnn_weight,
           gru_w_ih, gru_w_hh, gru_b_ih, gru_b_hh, gate_w, gate_b,
           lstm_w_ih, lstm_w_hh, lstm_b_ih, lstm_b_hh, dense_w, dense_b):
  N, _ = x.shape
  E = edge_index.shape[1]
  D = embed_w.shape[1]
  L = ggnn_weight.shape[0]
  LSTM = lstm_w_hh.shape[1]
  V = dense_w.shape[0]

  TILES = NC * NS
  P = ((N + TILES * CH - 1) // (TILES * CH)) * (TILES * CH)
  BN = P // 8
  EP = ((E + TILES * CH - 1) // (TILES * CH)) * (TILES * CH)

  xi = jnp.pad(x[:, 0], (0, P - N), mode="edge")
  src = edge_index[0]
  dst = edge_index[1]
  if EP != E:
    src = jnp.pad(src, (0, EP - E))
    dst = jnp.pad(dst, (0, EP - E), constant_values=P - 1)
  zrows = jnp.zeros((CH, D), jnp.float32)
  b_ih = gru_b_ih.reshape(1, 3 * D)
  b_hh = gru_b_hh.reshape(1, 3 * D)
  gb = gate_b.reshape(1, 1)
  lb = (lstm_b_ih + lstm_b_hh).reshape(1, 4 * LSTM)
  db = dense_b.reshape(1, V)

  h = _sc_embed(embed_w, xi, P, D)
  m = _tc_m0(h, ggnn_weight[0], P, D, BN)
  for i in range(L):
    part = _sc_scatter(m, src, dst, zrows, P, D)
    part = part.reshape(NC, P, D)
    h, m = _tc_step(part, h, gru_w_ih, gru_w_hh, b_ih, b_hh,
                    ggnn_weight[(i + 1) % L], P, D, BN)
  ht = _tc_pool_lstm(h, gate_w, gb, lstm_w_ih, lb, P, D, BN, N, LSTM)
  logits = _tc_dense(ht, dense_w, db, LSTM, V, 2048)
  return logits[:, None, :]
